# baseline (speedup 1.0000x reference)
import functools

import jax
import jax.numpy as jnp
from jax import lax
from jax.experimental import pallas as pl
from jax.experimental.pallas import tpu as pltpu

_N = 4194304
_ROWS = 512
_COLS = _N // _ROWS
_BR = 32
_GRID = _ROWS // _BR


def _tc_body(tab_ref, idx_ref, out_ref):
    i = idx_ref[...]
    t0 = tab_ref[0]
    t1 = tab_ref[1]
    t2 = tab_ref[2]
    t3 = tab_ref[3]
    lo = jnp.where(i == 0, t0, t1)
    hi = jnp.where(i == 2, t2, t3)
    out_ref[...] = jnp.where(i < 2, lo, hi)


_tc_gather = pl.pallas_call(
    _tc_body,
    grid=(_GRID,),
    in_specs=[
        pl.BlockSpec(memory_space=pltpu.SMEM),
        pl.BlockSpec((_BR, _COLS), lambda i: (i, 0)),
    ],
    out_specs=pl.BlockSpec((_BR, _COLS), lambda i: (i, 0)),
    out_shape=jax.ShapeDtypeStruct((_ROWS, _COLS), jnp.float32),
    compiler_params=pltpu.CompilerParams(
        dimension_semantics=("arbitrary",),
    ),
)


def kernel(supervision_weight, index, dummy, bin_num_examples):
    idx2d = index.reshape(_ROWS, _COLS)
    out = _tc_gather(bin_num_examples, idx2d)
    return out.reshape(_N)


# speedup vs baseline: 3.6268x; 3.6268x over previous
import functools

import jax
import jax.numpy as jnp
from jax import lax
from jax.experimental import pallas as pl
from jax.experimental.pallas import tpu as pltpu

_N = 4194304
_BLK = 262144
_GRID = _N // _BLK


def _tc_body(tab_ref, idx_ref, out_ref):
    i = idx_ref[...]
    t0 = tab_ref[0]
    t1 = tab_ref[1]
    t2 = tab_ref[2]
    t3 = tab_ref[3]
    lo = jnp.where(i == 0, t0, t1)
    hi = jnp.where(i == 2, t2, t3)
    out_ref[...] = jnp.where(i < 2, lo, hi)


_tc_gather = pl.pallas_call(
    _tc_body,
    grid=(_GRID,),
    in_specs=[
        pl.BlockSpec(memory_space=pltpu.SMEM),
        pl.BlockSpec((_BLK,), lambda i: (i,)),
    ],
    out_specs=pl.BlockSpec((_BLK,), lambda i: (i,)),
    out_shape=jax.ShapeDtypeStruct((_N,), jnp.float32),
    compiler_params=pltpu.CompilerParams(
        dimension_semantics=("arbitrary",),
    ),
)


def kernel(supervision_weight, index, dummy, bin_num_examples):
    return _tc_gather(bin_num_examples, index)
